# Initial kernel scaffold; baseline (speedup 1.0000x reference)
import jax
import jax.numpy as jnp
from jax.experimental import pallas as pl


def kernel(x, edge_index, edge_weight, enc_g, enc_mu, enc_sigma, enc_root, enc_bias, W_enc1, b_enc1, W_enc2, b_enc2, W_dec1, b_dec1, W_dec2, b_dec2, dec_g, dec_mu, dec_sigma, dec_root, dec_bias):
    raise NotImplementedError("write your pallas kernel here")


# speedup vs baseline: 10.0463x; 10.0463x over previous
import dataclasses
import functools

import jax
import jax.numpy as jnp
from jax import lax
from jax.experimental import pallas as pl
from jax.experimental.pallas import tpu as pltpu
from jax.experimental.pallas import tpu_sc as plsc

N = 10000
C = 8
E = 640000
K = 5
FFN = 200
BOT = 16

NW = 32
CHUNK = 80
EPW = E // NW
CPW = EPW // CHUNK
NBLK = NW * CPW
D = 48

BK = 3200
NKS = (N * C) // BK


def _elu(x):
    return jnp.where(x > 0, x, jnp.exp(jnp.minimum(x, 0.0)) - 1.0)


def _prologue_body(x_ref, g_ref, rootT_ref, b_ref, xg_ref, root_ref):
    x = x_ref[...]
    xg_ref[...] = jnp.dot(x, g_ref[...], preferred_element_type=jnp.float32)
    root_ref[...] = (
        jnp.dot(x, rootT_ref[...], preferred_element_type=jnp.float32)
        + b_ref[...]
    )


def _prologue(x, g_pad, rootT, bias2):
    return pl.pallas_call(
        _prologue_body,
        out_shape=(
            jax.ShapeDtypeStruct((N, D), jnp.float32),
            jax.ShapeDtypeStruct((N, C), jnp.float32),
        ),
    )(x, g_pad, rootT, bias2)


_NR = (N * C) // 128


def _combine_body(parts_ref, cnt_ref, root_ref, res_ref, o_ref, *, apply_elu):
    acc = parts_ref[0]
    cnt = cnt_ref[0]
    for i in range(1, NW):
        acc = acc + parts_ref[i]
        cnt = cnt + cnt_ref[i]
    inv = 1.0 / jnp.maximum(cnt, 1.0)
    lane = lax.broadcasted_iota(jnp.int32, (16, 128), 1)
    grp = lax.broadcasted_iota(jnp.int32, (16, 128), 0)
    rep = jnp.where(lane // C == grp, 1.0, 0.0).astype(jnp.float32)
    inv128 = jnp.dot(inv, rep, preferred_element_type=jnp.float32)
    pre = acc * inv128 + root_ref[...] + res_ref[...]
    o_ref[...] = _elu(pre) if apply_elu else pre


def _combine(parts, cnt, root, res, apply_elu):
    out = pl.pallas_call(
        functools.partial(_combine_body, apply_elu=apply_elu),
        out_shape=jax.ShapeDtypeStruct((_NR, 128), jnp.float32),
    )(
        parts.reshape(NW, _NR, 128),
        cnt.reshape(NW, _NR, 16),
        root.reshape(_NR, 128),
        res.reshape(_NR, 128),
    )
    return out.reshape(N, C)


def _gemv1_body(w_ref, x_ref, b_ref, o_ref):
    k = pl.program_id(0)

    @pl.when(k == 0)
    def _():
        o_ref[...] = jnp.zeros_like(o_ref)

    o_ref[...] += jnp.dot(
        w_ref[...], x_ref[...], preferred_element_type=jnp.float32
    )

    @pl.when(k == NKS - 1)
    def _():
        o_ref[...] = _elu(o_ref[...] + b_ref[...])


def _gemv1(w, flat, b2d):
    return pl.pallas_call(
        _gemv1_body,
        grid=(NKS,),
        in_specs=[
            pl.BlockSpec((FFN, BK), lambda k: (0, k)),
            pl.BlockSpec((BK, 1), lambda k: (k, 0)),
            pl.BlockSpec((FFN, 1), lambda k: (0, 0)),
        ],
        out_specs=pl.BlockSpec((FFN, 1), lambda k: (0, 0)),
        out_shape=jax.ShapeDtypeStruct((FFN, 1), jnp.float32),
    )(w, flat, b2d)


def _midfc_body(w2_ref, b2_ref, wd1_ref, bd1_ref, h1_ref, o_ref):
    z = (
        jnp.dot(w2_ref[...], h1_ref[...], preferred_element_type=jnp.float32)
        + b2_ref[...]
    )
    o_ref[...] = _elu(
        jnp.dot(wd1_ref[...], z, preferred_element_type=jnp.float32)
        + bd1_ref[...]
    )


def _midfc(w2, b2, wd1, bd1, h1):
    return pl.pallas_call(
        _midfc_body,
        out_shape=jax.ShapeDtypeStruct((FFN, 1), jnp.float32),
    )(w2, b2, wd1, bd1, h1)


def _gemv4_body(w_ref, d_ref, b_ref, o_ref):
    o_ref[...] = _elu(
        jnp.dot(w_ref[...], d_ref[...], preferred_element_type=jnp.float32)
        + b_ref[...]
    )


def _gemv4(w, d1, b2d):
    return pl.pallas_call(
        _gemv4_body,
        grid=(NKS,),
        in_specs=[
            pl.BlockSpec((BK, FFN), lambda m: (m, 0)),
            pl.BlockSpec((FFN, 1), lambda m: (0, 0)),
            pl.BlockSpec((BK, 1), lambda m: (m, 0)),
        ],
        out_specs=pl.BlockSpec((BK, 1), lambda m: (m, 0)),
        out_shape=jax.ShapeDtypeStruct((N * C, 1), jnp.float32),
    )(w, d1, b2d)


def _sc_conv(table, edata, params, with_count):
    mesh = plsc.VectorSubcoreMesh(core_axis_name="c", subcore_axis_name="s")

    out_type = [jax.ShapeDtypeStruct((NW, N * C), jnp.float32)]
    if with_count:
        out_type.append(jax.ShapeDtypeStruct((NW, N), jnp.float32))

    def body(table_h, edata_h, params_h, *rest):
        if with_count:
            parts_h, cnt_h = rest[0], rest[1]
            scr = rest[2:]
        else:
            parts_h = rest[0]
            cnt_h = None
            scr = rest[1:]
        agg_v, cnt_v, ed0_v, ed1_v, rows0_v, rows1_v, par_v, \
            se0, se1, sr0, sr1 = scr
        ed_bufs = (ed0_v, ed1_v)
        rows_bufs = (rows0_v, rows1_v)

        cid = lax.axis_index("c")
        sid = lax.axis_index("s")
        wid = sid * 2 + cid

        zero16 = jnp.zeros((16,), jnp.float32)

        @pl.loop(0, N * C, step=16, unroll=8)
        def _(i):
            agg_v[pl.ds(i, 16)] = zero16

        if with_count:
            @pl.loop(0, N, step=16, unroll=8)
            def _(i):
                cnt_v[pl.ds(i, 16)] = zero16

        pltpu.sync_copy(params_h, par_v)

        base_blk = wid * CPW
        sems_e = (se0, se1)
        sems_r = (sr0, sr1)

        def e_copy(j, b):
            return pltpu.make_async_copy(
                edata_h.at[base_blk + j], ed_bufs[b], sems_e[b]
            )

        def r_copy(b):
            return pltpu.make_async_copy(
                table_h.at[ed_bufs[b].at[0]], rows_bufs[b], sems_r[b]
            )

        e_copy(0, 0).start()
        e_copy(1, 1).start()
        e_copy(0, 0).wait()
        r_copy(0).start()

        iota16 = lax.iota(jnp.int32, 16)
        mus = [par_v[k] for k in range(K)]
        avs = [par_v[K + k] for k in range(K)]
        ones16 = jnp.full((16,), 1.0, jnp.float32)
        colvs = [
            [jnp.full((16,), k * C + c, jnp.int32) for c in range(C)]
            for k in range(K)
        ]

        def do_chunk(j, b):
            ob = 1 - b

            @pl.when(j + 1 < CPW)
            def _():
                e_copy(j + 1, ob).wait()
                r_copy(ob).start()

            r_copy(b).wait()

            ed = ed_bufs[b]
            dstvs = [ed[1, pl.ds(t * 16, 16)] for t in range(5)]
            wvs = [
                plsc.bitcast(ed[2, pl.ds(t * 16, 16)], jnp.float32)
                for t in range(5)
            ]

            @pl.when(j + 2 < CPW)
            def _():
                e_copy(j + 2, b).start()

            rows = rows_bufs[b]
            for t in range(5):
                wv = wvs[t]
                dstv = dstvs[t]
                gs = []
                for k in range(K):
                    d = wv - mus[k]
                    gs.append(jnp.exp(d * d * avs[k]))
                rid = iota16 + (t * 16)
                d8 = dstv * C
                for c in range(C):
                    acc = None
                    for k in range(K):
                        v = plsc.load_gather(rows, [rid, colvs[k][c]])
                        term = gs[k] * v
                        acc = term if acc is None else acc + term
                    plsc.addupdate_scatter(agg_v, [d8 + c], acc)
                if with_count:
                    plsc.addupdate_scatter(cnt_v, [dstv], ones16)

        @pl.loop(0, CPW, step=2)
        def _(i):
            do_chunk(i, 0)
            do_chunk(i + 1, 1)

        pltpu.sync_copy(agg_v, parts_h.at[wid])
        if with_count:
            pltpu.sync_copy(cnt_v, cnt_h.at[wid])

    scratch = [
        pltpu.VMEM((N * C,), jnp.float32),
        pltpu.VMEM((N,), jnp.float32),
        pltpu.VMEM((3, CHUNK), jnp.int32),
        pltpu.VMEM((3, CHUNK), jnp.int32),
        pltpu.VMEM((CHUNK, D), jnp.float32),
        pltpu.VMEM((CHUNK, D), jnp.float32),
        pltpu.VMEM((2 * K, 16), jnp.float32),
        pltpu.SemaphoreType.DMA,
        pltpu.SemaphoreType.DMA,
        pltpu.SemaphoreType.DMA,
        pltpu.SemaphoreType.DMA,
    ]

    cp = pltpu.CompilerParams()
    if "needs_layout_passes" in pltpu.CompilerParams.__dataclass_fields__:
        cp = dataclasses.replace(cp, needs_layout_passes=False)
    if "use_tc_tiling_on_sc" in pltpu.CompilerParams.__dataclass_fields__:
        cp = dataclasses.replace(cp, use_tc_tiling_on_sc=False)

    run = pl.kernel(
        body, out_type=tuple(out_type), mesh=mesh, scratch_types=scratch,
        compiler_params=cp,
    )
    return run(table, edata, params)


def _pack_params(mu, sigma):
    a = -0.5 / (1e-15 + sigma[:, 0] ** 2)
    m = mu[:, 0]
    both = jnp.concatenate([m, a], 0)
    return jnp.tile(both[:, None], (1, 16)).astype(jnp.float32)


def kernel(x, edge_index, edge_weight,
           enc_g, enc_mu, enc_sigma, enc_root, enc_bias,
           W_enc1, b_enc1, W_enc2, b_enc2,
           W_dec1, b_dec1, W_dec2, b_dec2,
           dec_g, dec_mu, dec_sigma, dec_root, dec_bias):
    src = edge_index[0].astype(jnp.int32)
    dst = edge_index[1].astype(jnp.int32)
    wbits = lax.bitcast_convert_type(
        edge_weight[:, 0].astype(jnp.float32), jnp.int32
    )
    edata = (
        jnp.stack([src, dst, wbits], 0)
        .reshape(3, NBLK, CHUNK)
        .transpose(1, 0, 2)
    )

    pad = ((0, 0), (0, D - K * C))
    params1 = _pack_params(enc_mu, enc_sigma)
    params2 = _pack_params(dec_mu, dec_sigma)

    xg1, root1 = _prologue(
        x, jnp.pad(enc_g, pad), enc_root.T, enc_bias[None, :]
    )
    parts1, cnt = _sc_conv(xg1, edata, params1, with_count=True)
    xe = _combine(parts1, cnt, root1, x, apply_elu=True)

    flat = xe.reshape(N * C, 1)
    h1 = _gemv1(W_enc1, flat, b_enc1[:, None])
    d1 = _midfc(W_enc2, b_enc2[:, None], W_dec1, b_dec1[:, None], h1)
    xd = _gemv4(W_dec2, d1, b_dec2[:, None]).reshape(N, C)

    xg2, root2 = _prologue(
        xd, jnp.pad(dec_g, pad), dec_root.T, dec_bias[None, :]
    )
    (parts2,) = _sc_conv(xg2, edata, params2, with_count=False)
    out = _combine(parts2, cnt, root2, xd, apply_elu=False)
    return out
